# baseline (speedup 1.0000x reference)
import functools

import jax
import jax.numpy as jnp
import numpy as np
from jax import lax
from jax.experimental import pallas as pl
from jax.experimental.pallas import tpu as pltpu
from jax.experimental.pallas import tpu_sc as plsc

B, H, W, C = 4, 224, 224, 96
POOL = 8
HP, WP = H // POOL, W // POOL
NPIX = B * H * W


def _keys_cubic(x):
    x = np.abs(x)
    out = ((1.5 * x - 2.5) * x) * x + 1.0
    out = np.where(x >= 1.0, ((-0.5 * x + 2.5) * x - 4.0) * x + 2.0, out)
    return np.where(x >= 2.0, 0.0, out)


def _resize_mat(n_in, n_out):
    scale = n_out / n_in
    sample = (np.arange(n_out) + 0.5) / scale - 0.5
    w = _keys_cubic(sample[:, None] - np.arange(n_in)[None, :])
    w = w / w.sum(axis=1, keepdims=True)
    return w.astype(np.float32)


_RH = _resize_mat(HP, H)
_RWT = np.ascontiguousarray(_resize_mat(WP, W).T)

_PMAT = np.repeat(np.eye(WP, dtype=np.float32), POOL, axis=1) / (POOL * POOL)


CP = 128


def _pool_body(x_hbm, p_ref, o_ref, xp_ref, xv, sem):
    b = pl.program_id(0)
    i = pl.program_id(1)
    cp = pltpu.make_async_copy(x_hbm.at[b, pl.ds(i * POOL, POOL)], xv, sem)
    cp.start()
    cp.wait()
    x = xv[...]
    s = jnp.sum(x, axis=0)
    o_ref[0, 0] = jnp.dot(p_ref[...], s, preferred_element_type=jnp.float32, precision=lax.Precision.HIGHEST)
    xp_ref[...] = jnp.concatenate(
        [x, jnp.zeros((POOL, W, CP - C), jnp.float32)],
        axis=-1).reshape(POOL * W, CP)


def _pool(x):
    return pl.pallas_call(
        _pool_body,
        grid=(B, HP),
        in_specs=[
            pl.BlockSpec(memory_space=pltpu.MemorySpace.HBM),
            pl.BlockSpec((WP, W), lambda b, i: (0, 0)),
        ],
        out_specs=[
            pl.BlockSpec((1, 1, WP, C), lambda b, i: (b, i, 0, 0)),
            pl.BlockSpec((POOL * W, CP), lambda b, i: (b * HP + i, 0)),
        ],
        out_shape=[
            jax.ShapeDtypeStruct((B, HP, WP, C), jnp.float32),
            jax.ShapeDtypeStruct((NPIX, CP), jnp.float32),
        ],
        scratch_shapes=[
            pltpu.VMEM((POOL, W, C), jnp.float32),
            pltpu.SemaphoreType.DMA,
        ],
    )(x, _PMAT)


def _coord_body(xp_ref, wbx_ref, wby_ref, bl_ref, rh_ref, rwt_ref,
                ia_ref, ib_ref, ic_ref, id_ref,
                wa_ref, wb_ref, wc_ref, wd_ref):
    b = pl.program_id(0)
    xp = xp_ref[0]
    rh = rh_ref[...]
    rwt = rwt_ref[...]

    pre_x = jnp.dot(xp, wbx_ref[...], preferred_element_type=jnp.float32, precision=lax.Precision.HIGHEST)
    vx = jnp.tanh(pre_x + bl_ref[0, 0]) * 0.1
    pre_y = jnp.dot(xp, wby_ref[...], preferred_element_type=jnp.float32, precision=lax.Precision.HIGHEST)
    vy = jnp.tanh(pre_y + bl_ref[0, 1]) * 0.1

    vrx = jnp.dot(jnp.dot(rh, vx, preferred_element_type=jnp.float32, precision=lax.Precision.HIGHEST), rwt,
                  preferred_element_type=jnp.float32, precision=lax.Precision.HIGHEST)
    vry = jnp.dot(jnp.dot(rh, vy, preferred_element_type=jnp.float32, precision=lax.Precision.HIGHEST), rwt,
                  preferred_element_type=jnp.float32, precision=lax.Precision.HIGHEST)

    gx = lax.broadcasted_iota(jnp.int32, (H, W), 1).astype(jnp.float32) \
        * (2.0 / (W - 1)) - 1.0
    gy = lax.broadcasted_iota(jnp.int32, (H, W), 0).astype(jnp.float32) \
        * (2.0 / (H - 1)) - 1.0

    px = (vrx + gx + 1.0) * (float(W) / 2.0)
    py = (vry + gy + 1.0) * (float(H) / 2.0)

    x0i = jnp.floor(px).astype(jnp.int32)
    y0i = jnp.floor(py).astype(jnp.int32)
    x0 = jnp.clip(x0i, 0, W - 1)
    x1 = jnp.clip(x0i + 1, 0, W - 1)
    y0 = jnp.clip(y0i, 0, H - 1)
    y1 = jnp.clip(y0i + 1, 0, H - 1)

    x0f = x0.astype(jnp.float32)
    x1f = x1.astype(jnp.float32)
    y0f = y0.astype(jnp.float32)
    y1f = y1.astype(jnp.float32)

    base = b * (H * W)
    ia_ref[0] = base + y0 * W + x0
    ib_ref[0] = base + y1 * W + x0
    ic_ref[0] = base + y0 * W + x1
    id_ref[0] = base + y1 * W + x1

    wa_ref[0] = (x1f - px) * (y1f - py)
    wb_ref[0] = (x1f - px) * (py - y0f)
    wc_ref[0] = (px - x0f) * (y1f - py)
    wd_ref[0] = (px - x0f) * (py - y0f)


def _coords(xp2, wbx, wby, bl):
    full = lambda shape: pl.BlockSpec(shape, lambda b: (0, 0))
    ospec = pl.BlockSpec((1, H, W), lambda b: (b, 0, 0))
    oshape_i = jax.ShapeDtypeStruct((B, H, W), jnp.int32)
    oshape_f = jax.ShapeDtypeStruct((B, H, W), jnp.float32)
    return pl.pallas_call(
        _coord_body,
        grid=(B,),
        in_specs=[
            pl.BlockSpec((1, HP, WP * C), lambda b: (b, 0, 0)),
            full((WP * C, WP)),
            full((WP * C, WP)),
            full((1, 2)),
            full((H, HP)),
            full((WP, W)),
        ],
        out_specs=[ospec] * 8,
        out_shape=[oshape_i] * 4 + [oshape_f] * 4,
    )(xp2, wbx, wby, bl, _RH, _RWT)


NC, NS, L = 2, 16, 16
NW = NC * NS
PIX_PER_TILE = NPIX // NW
CHUNK = 64
NCHUNK = PIX_PER_TILE // CHUNK
NPAIR = NCHUNK // 2
CVEC = C // L


def _sc_body(xflat, ia, ib, ic, idd, wa, wb, wc, wd, out3,
             iav, ibv, icv, idv,
             ra0, rb0, rc0, rd0, wv0, ra1, rb1, rc1, rd1, wv1, ov0, ov1,
             sg0, sg1, so0, so1):
    wid = lax.axis_index("s") * NC + lax.axis_index("c")
    base0 = wid * PIX_PER_TILE

    tile = pl.ds(base0, PIX_PER_TILE)
    pltpu.sync_copy(ia.at[tile], iav)
    pltpu.sync_copy(ib.at[tile], ibv)
    pltpu.sync_copy(ic.at[tile], icv)
    pltpu.sync_copy(idd.at[tile], idv)

    slots = ((ra0, rb0, rc0, rd0, wv0, ov0, sg0, so0),
             (ra1, rb1, rc1, rd1, wv1, ov1, sg1, so1))

    bimg = wid // (NW // B)
    ibase = base0 - bimg * (H * W)

    def fire(g, slot):
        ra, rb, rc, rd, wv, _, sg, _ = slots[slot]
        sl = pl.ds(g * CHUNK, CHUNK)
        gsl = pl.ds(base0 + g * CHUNK, CHUNK)
        pltpu.async_copy(xflat.at[iav.at[sl]], ra, sg)
        pltpu.async_copy(xflat.at[ibv.at[sl]], rb, sg)
        pltpu.async_copy(xflat.at[icv.at[sl]], rc, sg)
        pltpu.async_copy(xflat.at[idv.at[sl]], rd, sg)
        pltpu.async_copy(wa.at[gsl], wv.at[0], sg)
        pltpu.async_copy(wb.at[gsl], wv.at[1], sg)
        pltpu.async_copy(wc.at[gsl], wv.at[2], sg)
        pltpu.async_copy(wd.at[gsl], wv.at[3], sg)

    def wait_gathers(slot):
        ra, rb, rc, rd, wv, _, sg, _ = slots[slot]
        dummy = xflat.at[pl.ds(0, CHUNK)]
        for dst in (ra, rb, rc, rd):
            pltpu.make_async_copy(dummy, dst, sg).wait()
        wdummy = wa.at[pl.ds(0, CHUNK)]
        for k in range(4):
            pltpu.make_async_copy(wdummy, wv.at[k], sg).wait()

    def drain_out(slot):
        ov, so = slots[slot][5], slots[slot][7]
        pltpu.make_async_copy(ov, out3.at[0, pl.ds(0, CHUNK)], so).wait()

    def compute_and_store(g, slot):
        ra, rb, rc, rd, wv, ov, _, so = slots[slot]
        off = g * CHUNK

        def grp(q, c2):
            woff = q * L
            wa16 = wv[0, pl.ds(woff, L)]
            wb16 = wv[1, pl.ds(woff, L)]
            wc16 = wv[2, pl.ds(woff, L)]
            wd16 = wv[3, pl.ds(woff, L)]
            for k in range(L):
                p = q * L + k
                w0, w1, w2, w3 = wa16[k], wb16[k], wc16[k], wd16[k]
                for j in range(CVEC):
                    s = pl.ds(j * L, L)
                    ov[p, s] = (ra[p, s] * w0 + rb[p, s] * w1
                                + rc[p, s] * w2 + rd[p, s] * w3)
            return c2

        lax.fori_loop(0, CHUNK // L, grp, 0)
        pltpu.async_copy(ov, out3.at[bimg, pl.ds(ibase + off, CHUNK)], so)

    fire(0, 0)

    def pair(i, carry):
        g0 = i * 2
        fire(g0 + 1, 1)
        wait_gathers(0)

        @pl.when(i > 0)
        def _():
            drain_out(0)

        compute_and_store(g0, 0)

        @pl.when(i < NPAIR - 1)
        def _():
            fire(g0 + 2, 0)

        wait_gathers(1)

        @pl.when(i > 0)
        def _():
            drain_out(1)

        compute_and_store(g0 + 1, 1)
        return carry

    lax.fori_loop(0, NPAIR, pair, 0)
    drain_out(0)
    drain_out(1)


@functools.partial(jax.jit, static_argnums=())
def _sc_gather(xflat, ia, ib, ic, idd, wa, wb, wc, wd):
    mesh = plsc.VectorSubcoreMesh(core_axis_name="c", subcore_axis_name="s")
    kern = functools.partial(
        pl.kernel,
        mesh=mesh,
        compiler_params=pltpu.CompilerParams(use_tc_tiling_on_sc=True),
        out_type=jax.ShapeDtypeStruct((B, H * W, C), jnp.float32),
        scratch_types=[
            pltpu.VMEM((PIX_PER_TILE,), jnp.int32),
            pltpu.VMEM((PIX_PER_TILE,), jnp.int32),
            pltpu.VMEM((PIX_PER_TILE,), jnp.int32),
            pltpu.VMEM((PIX_PER_TILE,), jnp.int32),
            pltpu.VMEM((CHUNK, CP), jnp.float32),
            pltpu.VMEM((CHUNK, CP), jnp.float32),
            pltpu.VMEM((CHUNK, CP), jnp.float32),
            pltpu.VMEM((CHUNK, CP), jnp.float32),
            pltpu.VMEM((4, CHUNK), jnp.float32),
            pltpu.VMEM((CHUNK, CP), jnp.float32),
            pltpu.VMEM((CHUNK, CP), jnp.float32),
            pltpu.VMEM((CHUNK, CP), jnp.float32),
            pltpu.VMEM((CHUNK, CP), jnp.float32),
            pltpu.VMEM((4, CHUNK), jnp.float32),
            pltpu.VMEM((CHUNK, C), jnp.float32),
            pltpu.VMEM((CHUNK, C), jnp.float32),
            pltpu.SemaphoreType.DMA,
            pltpu.SemaphoreType.DMA,
            pltpu.SemaphoreType.DMA,
            pltpu.SemaphoreType.DMA,
        ],
    )(_sc_body)
    return kern(xflat, ia, ib, ic, idd, wa, wb, wc, wd)


def kernel(X, W_loc, b_loc):
    xp, xpad = _pool(X)
    xp2 = xp.reshape(B, HP, WP * C)
    eye = jnp.eye(WP, dtype=jnp.float32)
    wbx = jnp.kron(eye, W_loc[:, 0:1])
    wby = jnp.kron(eye, W_loc[:, 1:2])
    bl = b_loc.reshape(1, 2)
    ia, ib, ic, idd, wa, wb, wc, wd = _coords(xp2, wbx, wby, bl)
    out3 = _sc_gather(
        xpad,
        ia.reshape(-1), ib.reshape(-1), ic.reshape(-1), idd.reshape(-1),
        wa.reshape(-1), wb.reshape(-1), wc.reshape(-1), wd.reshape(-1),
    )
    return out3.reshape(B, H, W, C)


# speedup vs baseline: 1.1750x; 1.1750x over previous
import functools

import jax
import jax.numpy as jnp
import numpy as np
from jax import lax
from jax.experimental import pallas as pl
from jax.experimental.pallas import tpu as pltpu
from jax.experimental.pallas import tpu_sc as plsc

B, H, W, C = 4, 224, 224, 96
POOL = 8
HP, WP = H // POOL, W // POOL
NPIX = B * H * W


def _keys_cubic(x):
    x = np.abs(x)
    out = ((1.5 * x - 2.5) * x) * x + 1.0
    out = np.where(x >= 1.0, ((-0.5 * x + 2.5) * x - 4.0) * x + 2.0, out)
    return np.where(x >= 2.0, 0.0, out)


def _resize_mat(n_in, n_out):
    scale = n_out / n_in
    sample = (np.arange(n_out) + 0.5) / scale - 0.5
    w = _keys_cubic(sample[:, None] - np.arange(n_in)[None, :])
    w = w / w.sum(axis=1, keepdims=True)
    return w.astype(np.float32)


_RH = _resize_mat(HP, H)
_RWT = np.ascontiguousarray(_resize_mat(WP, W).T)

_PMAT = np.repeat(np.eye(WP, dtype=np.float32), POOL, axis=1) / (POOL * POOL)


CP = 128


NSTEP = B * HP


def _pool_body(x_hbm, p_ref, o_ref, xp_ref, xv0, xv1, sem0, sem1):
    b = pl.program_id(0)
    i = pl.program_id(1)
    k = b * HP + i
    par = lax.rem(k, 2)

    def cp_for(step, xv, sem):
        bb = step // HP
        ii = step - bb * HP
        return pltpu.make_async_copy(
            x_hbm.at[bb, pl.ds(ii * POOL, POOL)], xv, sem)

    @pl.when(k == 0)
    def _():
        cp_for(k, xv0, sem0).start()

    @pl.when(jnp.logical_and(k + 1 < NSTEP, par == 0))
    def _():
        cp_for(k + 1, xv1, sem1).start()

    @pl.when(jnp.logical_and(k + 1 < NSTEP, par == 1))
    def _():
        cp_for(k + 1, xv0, sem0).start()

    def finish(xv, sem):
        cp_for(k, xv, sem).wait()
        x = xv[...]
        s = jnp.sum(x, axis=0)
        o_ref[0, 0] = jnp.dot(p_ref[...], s,
                              preferred_element_type=jnp.float32,
                              precision=lax.Precision.HIGHEST)
        xp_ref[...] = jnp.concatenate(
            [x, jnp.zeros((POOL, W, CP - C), jnp.float32)],
            axis=-1).reshape(POOL * W, CP)

    @pl.when(par == 0)
    def _():
        finish(xv0, sem0)

    @pl.when(par == 1)
    def _():
        finish(xv1, sem1)


def _pool(x):
    return pl.pallas_call(
        _pool_body,
        grid=(B, HP),
        in_specs=[
            pl.BlockSpec(memory_space=pltpu.MemorySpace.HBM),
            pl.BlockSpec((WP, W), lambda b, i: (0, 0)),
        ],
        out_specs=[
            pl.BlockSpec((1, 1, WP, C), lambda b, i: (b, i, 0, 0)),
            pl.BlockSpec((POOL * W, CP), lambda b, i: (b * HP + i, 0)),
        ],
        out_shape=[
            jax.ShapeDtypeStruct((B, HP, WP, C), jnp.float32),
            jax.ShapeDtypeStruct((NPIX, CP), jnp.float32),
        ],
        scratch_shapes=[
            pltpu.VMEM((POOL, W, C), jnp.float32),
            pltpu.VMEM((POOL, W, C), jnp.float32),
            pltpu.SemaphoreType.DMA,
            pltpu.SemaphoreType.DMA,
        ],
    )(x, _PMAT)


def _coord_body(xp_ref, wbx_ref, wby_ref, bl_ref, rh_ref, rwt_ref,
                ia_ref, ib_ref, ic_ref, id_ref,
                wa_ref, wb_ref, wc_ref, wd_ref):
    b = pl.program_id(0)
    xp = xp_ref[0]
    rh = rh_ref[...]
    rwt = rwt_ref[...]

    pre_x = jnp.dot(xp, wbx_ref[...], preferred_element_type=jnp.float32, precision=lax.Precision.HIGHEST)
    vx = jnp.tanh(pre_x + bl_ref[0, 0]) * 0.1
    pre_y = jnp.dot(xp, wby_ref[...], preferred_element_type=jnp.float32, precision=lax.Precision.HIGHEST)
    vy = jnp.tanh(pre_y + bl_ref[0, 1]) * 0.1

    vrx = jnp.dot(jnp.dot(rh, vx, preferred_element_type=jnp.float32, precision=lax.Precision.HIGHEST), rwt,
                  preferred_element_type=jnp.float32, precision=lax.Precision.HIGHEST)
    vry = jnp.dot(jnp.dot(rh, vy, preferred_element_type=jnp.float32, precision=lax.Precision.HIGHEST), rwt,
                  preferred_element_type=jnp.float32, precision=lax.Precision.HIGHEST)

    gx = lax.broadcasted_iota(jnp.int32, (H, W), 1).astype(jnp.float32) \
        * (2.0 / (W - 1)) - 1.0
    gy = lax.broadcasted_iota(jnp.int32, (H, W), 0).astype(jnp.float32) \
        * (2.0 / (H - 1)) - 1.0

    px = (vrx + gx + 1.0) * (float(W) / 2.0)
    py = (vry + gy + 1.0) * (float(H) / 2.0)

    x0i = jnp.floor(px).astype(jnp.int32)
    y0i = jnp.floor(py).astype(jnp.int32)
    x0 = jnp.clip(x0i, 0, W - 1)
    x1 = jnp.clip(x0i + 1, 0, W - 1)
    y0 = jnp.clip(y0i, 0, H - 1)
    y1 = jnp.clip(y0i + 1, 0, H - 1)

    x0f = x0.astype(jnp.float32)
    x1f = x1.astype(jnp.float32)
    y0f = y0.astype(jnp.float32)
    y1f = y1.astype(jnp.float32)

    base = b * (H * W)
    ia_ref[0] = base + y0 * W + x0
    ib_ref[0] = base + y1 * W + x0
    ic_ref[0] = base + y0 * W + x1
    id_ref[0] = base + y1 * W + x1

    wa_ref[0] = (x1f - px) * (y1f - py)
    wb_ref[0] = (x1f - px) * (py - y0f)
    wc_ref[0] = (px - x0f) * (y1f - py)
    wd_ref[0] = (px - x0f) * (py - y0f)


def _coords(xp2, wbx, wby, bl):
    full = lambda shape: pl.BlockSpec(shape, lambda b: (0, 0))
    ospec = pl.BlockSpec((1, H, W), lambda b: (b, 0, 0))
    oshape_i = jax.ShapeDtypeStruct((B, H, W), jnp.int32)
    oshape_f = jax.ShapeDtypeStruct((B, H, W), jnp.float32)
    return pl.pallas_call(
        _coord_body,
        grid=(B,),
        in_specs=[
            pl.BlockSpec((1, HP, WP * C), lambda b: (b, 0, 0)),
            full((WP * C, WP)),
            full((WP * C, WP)),
            full((1, 2)),
            full((H, HP)),
            full((WP, W)),
        ],
        out_specs=[ospec] * 8,
        out_shape=[oshape_i] * 4 + [oshape_f] * 4,
    )(xp2, wbx, wby, bl, _RH, _RWT)


NC, NS, L = 2, 16, 16
NW = NC * NS
PIX_PER_TILE = NPIX // NW
CHUNK = 64
NCHUNK = PIX_PER_TILE // CHUNK
NPAIR = NCHUNK // 2
CVEC = C // L


def _sc_body(xflat, ia, ib, ic, idd, wa, wb, wc, wd, out3,
             iav, ibv, icv, idv,
             ra0, rb0, rc0, rd0, wv0, ra1, rb1, rc1, rd1, wv1, ov0, ov1,
             sg0, sg1, so0, so1):
    wid = lax.axis_index("s") * NC + lax.axis_index("c")
    base0 = wid * PIX_PER_TILE

    tile = pl.ds(base0, PIX_PER_TILE)
    pltpu.sync_copy(ia.at[tile], iav)
    pltpu.sync_copy(ib.at[tile], ibv)
    pltpu.sync_copy(ic.at[tile], icv)
    pltpu.sync_copy(idd.at[tile], idv)

    slots = ((ra0, rb0, rc0, rd0, wv0, ov0, sg0, so0),
             (ra1, rb1, rc1, rd1, wv1, ov1, sg1, so1))

    bimg = wid // (NW // B)
    ibase = base0 - bimg * (H * W)

    def fire(g, slot):
        ra, rb, rc, rd, wv, _, sg, _ = slots[slot]
        sl = pl.ds(g * CHUNK, CHUNK)
        gsl = pl.ds(base0 + g * CHUNK, CHUNK)
        pltpu.async_copy(xflat.at[iav.at[sl]], ra, sg)
        pltpu.async_copy(xflat.at[ibv.at[sl]], rb, sg)
        pltpu.async_copy(xflat.at[icv.at[sl]], rc, sg)
        pltpu.async_copy(xflat.at[idv.at[sl]], rd, sg)
        pltpu.async_copy(wa.at[gsl], wv.at[0], sg)
        pltpu.async_copy(wb.at[gsl], wv.at[1], sg)
        pltpu.async_copy(wc.at[gsl], wv.at[2], sg)
        pltpu.async_copy(wd.at[gsl], wv.at[3], sg)

    def wait_gathers(slot):
        ra, rb, rc, rd, wv, _, sg, _ = slots[slot]
        dummy = xflat.at[pl.ds(0, CHUNK)]
        for dst in (ra, rb, rc, rd):
            pltpu.make_async_copy(dummy, dst, sg).wait()
        wdummy = wa.at[pl.ds(0, CHUNK)]
        for k in range(4):
            pltpu.make_async_copy(wdummy, wv.at[k], sg).wait()

    def drain_out(slot):
        ov, so = slots[slot][5], slots[slot][7]
        pltpu.make_async_copy(ov, out3.at[0, pl.ds(0, CHUNK)], so).wait()

    def compute_and_store(g, slot):
        ra, rb, rc, rd, wv, ov, _, so = slots[slot]
        off = g * CHUNK

        def grp(q, c2):
            woff = q * L
            wa16 = wv[0, pl.ds(woff, L)]
            wb16 = wv[1, pl.ds(woff, L)]
            wc16 = wv[2, pl.ds(woff, L)]
            wd16 = wv[3, pl.ds(woff, L)]
            for k in range(L):
                p = q * L + k
                w0, w1, w2, w3 = wa16[k], wb16[k], wc16[k], wd16[k]
                for j in range(CVEC):
                    s = pl.ds(j * L, L)
                    ov[p, s] = (ra[p, s] * w0 + rb[p, s] * w1
                                + rc[p, s] * w2 + rd[p, s] * w3)
            return c2

        lax.fori_loop(0, CHUNK // L, grp, 0)
        pltpu.async_copy(ov, out3.at[bimg, pl.ds(ibase + off, CHUNK)], so)

    fire(0, 0)

    def pair(i, carry):
        g0 = i * 2
        fire(g0 + 1, 1)
        wait_gathers(0)

        @pl.when(i > 0)
        def _():
            drain_out(0)

        compute_and_store(g0, 0)

        @pl.when(i < NPAIR - 1)
        def _():
            fire(g0 + 2, 0)

        wait_gathers(1)

        @pl.when(i > 0)
        def _():
            drain_out(1)

        compute_and_store(g0 + 1, 1)
        return carry

    lax.fori_loop(0, NPAIR, pair, 0)
    drain_out(0)
    drain_out(1)


@functools.partial(jax.jit, static_argnums=())
def _sc_gather(xflat, ia, ib, ic, idd, wa, wb, wc, wd):
    mesh = plsc.VectorSubcoreMesh(core_axis_name="c", subcore_axis_name="s")
    kern = functools.partial(
        pl.kernel,
        mesh=mesh,
        compiler_params=pltpu.CompilerParams(use_tc_tiling_on_sc=True),
        out_type=jax.ShapeDtypeStruct((B, H * W, C), jnp.float32),
        scratch_types=[
            pltpu.VMEM((PIX_PER_TILE,), jnp.int32),
            pltpu.VMEM((PIX_PER_TILE,), jnp.int32),
            pltpu.VMEM((PIX_PER_TILE,), jnp.int32),
            pltpu.VMEM((PIX_PER_TILE,), jnp.int32),
            pltpu.VMEM((CHUNK, CP), jnp.float32),
            pltpu.VMEM((CHUNK, CP), jnp.float32),
            pltpu.VMEM((CHUNK, CP), jnp.float32),
            pltpu.VMEM((CHUNK, CP), jnp.float32),
            pltpu.VMEM((4, CHUNK), jnp.float32),
            pltpu.VMEM((CHUNK, CP), jnp.float32),
            pltpu.VMEM((CHUNK, CP), jnp.float32),
            pltpu.VMEM((CHUNK, CP), jnp.float32),
            pltpu.VMEM((CHUNK, CP), jnp.float32),
            pltpu.VMEM((4, CHUNK), jnp.float32),
            pltpu.VMEM((CHUNK, C), jnp.float32),
            pltpu.VMEM((CHUNK, C), jnp.float32),
            pltpu.SemaphoreType.DMA,
            pltpu.SemaphoreType.DMA,
            pltpu.SemaphoreType.DMA,
            pltpu.SemaphoreType.DMA,
        ],
    )(_sc_body)
    return kern(xflat, ia, ib, ic, idd, wa, wb, wc, wd)


def kernel(X, W_loc, b_loc):
    xp, xpad = _pool(X)
    xp2 = xp.reshape(B, HP, WP * C)
    eye = jnp.eye(WP, dtype=jnp.float32)
    wbx = jnp.kron(eye, W_loc[:, 0:1])
    wby = jnp.kron(eye, W_loc[:, 1:2])
    bl = b_loc.reshape(1, 2)
    ia, ib, ic, idd, wa, wb, wc, wd = _coords(xp2, wbx, wby, bl)
    out3 = _sc_gather(
        xpad,
        ia.reshape(-1), ib.reshape(-1), ic.reshape(-1), idd.reshape(-1),
        wa.reshape(-1), wb.reshape(-1), wc.reshape(-1), wd.reshape(-1),
    )
    return out3.reshape(B, H, W, C)


# speedup vs baseline: 1.1750x; 1.0001x over previous
import functools

import jax
import jax.numpy as jnp
import numpy as np
from jax import lax
from jax.experimental import pallas as pl
from jax.experimental.pallas import tpu as pltpu
from jax.experimental.pallas import tpu_sc as plsc

B, H, W, C = 4, 224, 224, 96
POOL = 8
HP, WP = H // POOL, W // POOL
NPIX = B * H * W


def _keys_cubic(x):
    x = np.abs(x)
    out = ((1.5 * x - 2.5) * x) * x + 1.0
    out = np.where(x >= 1.0, ((-0.5 * x + 2.5) * x - 4.0) * x + 2.0, out)
    return np.where(x >= 2.0, 0.0, out)


def _resize_mat(n_in, n_out):
    scale = n_out / n_in
    sample = (np.arange(n_out) + 0.5) / scale - 0.5
    w = _keys_cubic(sample[:, None] - np.arange(n_in)[None, :])
    w = w / w.sum(axis=1, keepdims=True)
    return w.astype(np.float32)


_RH = _resize_mat(HP, H)
_RWT = np.ascontiguousarray(_resize_mat(WP, W).T)

_PMAT = np.repeat(np.eye(WP, dtype=np.float32), POOL, axis=1) / (POOL * POOL)


CP = 128


def _pool_body(x_ref, p_ref, o_ref, xp_ref):
    x = x_ref[0]
    s = jnp.sum(x, axis=0)
    o_ref[0, 0] = jnp.dot(p_ref[...], s, preferred_element_type=jnp.float32, precision=lax.Precision.HIGHEST)
    xp_ref[...] = jnp.concatenate(
        [x, jnp.zeros((POOL, W, CP - C), jnp.float32)],
        axis=-1).reshape(POOL * W, CP)


def _pool(x):
    return pl.pallas_call(
        _pool_body,
        grid=(B, HP),
        in_specs=[
            pl.BlockSpec((1, POOL, W, C), lambda b, i: (b, i, 0, 0)),
            pl.BlockSpec((WP, W), lambda b, i: (0, 0)),
        ],
        out_specs=[
            pl.BlockSpec((1, 1, WP, C), lambda b, i: (b, i, 0, 0)),
            pl.BlockSpec((POOL * W, CP), lambda b, i: (b * HP + i, 0)),
        ],
        out_shape=[
            jax.ShapeDtypeStruct((B, HP, WP, C), jnp.float32),
            jax.ShapeDtypeStruct((NPIX, CP), jnp.float32),
        ],
    )(x, _PMAT)


def _coord_body(xp_ref, wbx_ref, wby_ref, bl_ref, rh_ref, rwt_ref,
                ia_ref, ib_ref, ic_ref, id_ref,
                wa_ref, wb_ref, wc_ref, wd_ref):
    b = pl.program_id(0)
    xp = xp_ref[0]
    rh = rh_ref[...]
    rwt = rwt_ref[...]

    pre_x = jnp.dot(xp, wbx_ref[...], preferred_element_type=jnp.float32, precision=lax.Precision.HIGHEST)
    vx = jnp.tanh(pre_x + bl_ref[0, 0]) * 0.1
    pre_y = jnp.dot(xp, wby_ref[...], preferred_element_type=jnp.float32, precision=lax.Precision.HIGHEST)
    vy = jnp.tanh(pre_y + bl_ref[0, 1]) * 0.1

    vrx = jnp.dot(jnp.dot(rh, vx, preferred_element_type=jnp.float32, precision=lax.Precision.HIGHEST), rwt,
                  preferred_element_type=jnp.float32, precision=lax.Precision.HIGHEST)
    vry = jnp.dot(jnp.dot(rh, vy, preferred_element_type=jnp.float32, precision=lax.Precision.HIGHEST), rwt,
                  preferred_element_type=jnp.float32, precision=lax.Precision.HIGHEST)

    gx = lax.broadcasted_iota(jnp.int32, (H, W), 1).astype(jnp.float32) \
        * (2.0 / (W - 1)) - 1.0
    gy = lax.broadcasted_iota(jnp.int32, (H, W), 0).astype(jnp.float32) \
        * (2.0 / (H - 1)) - 1.0

    px = (vrx + gx + 1.0) * (float(W) / 2.0)
    py = (vry + gy + 1.0) * (float(H) / 2.0)

    x0i = jnp.floor(px).astype(jnp.int32)
    y0i = jnp.floor(py).astype(jnp.int32)
    x0 = jnp.clip(x0i, 0, W - 1)
    x1 = jnp.clip(x0i + 1, 0, W - 1)
    y0 = jnp.clip(y0i, 0, H - 1)
    y1 = jnp.clip(y0i + 1, 0, H - 1)

    x0f = x0.astype(jnp.float32)
    x1f = x1.astype(jnp.float32)
    y0f = y0.astype(jnp.float32)
    y1f = y1.astype(jnp.float32)

    base = b * (H * W)
    ia_ref[0] = base + y0 * W + x0
    ib_ref[0] = base + y1 * W + x0
    ic_ref[0] = base + y0 * W + x1
    id_ref[0] = base + y1 * W + x1

    wa_ref[0] = (x1f - px) * (y1f - py)
    wb_ref[0] = (x1f - px) * (py - y0f)
    wc_ref[0] = (px - x0f) * (y1f - py)
    wd_ref[0] = (px - x0f) * (py - y0f)


def _coords(xp2, wbx, wby, bl):
    full = lambda shape: pl.BlockSpec(shape, lambda b: (0, 0))
    ospec = pl.BlockSpec((1, H, W), lambda b: (b, 0, 0))
    oshape_i = jax.ShapeDtypeStruct((B, H, W), jnp.int32)
    oshape_f = jax.ShapeDtypeStruct((B, H, W), jnp.float32)
    return pl.pallas_call(
        _coord_body,
        grid=(B,),
        in_specs=[
            pl.BlockSpec((1, HP, WP * C), lambda b: (b, 0, 0)),
            full((WP * C, WP)),
            full((WP * C, WP)),
            full((1, 2)),
            full((H, HP)),
            full((WP, W)),
        ],
        out_specs=[ospec] * 8,
        out_shape=[oshape_i] * 4 + [oshape_f] * 4,
    )(xp2, wbx, wby, bl, _RH, _RWT)


NC, NS, L = 2, 16, 16
NW = NC * NS
PIX_PER_TILE = NPIX // NW
CHUNK = 64
NCHUNK = PIX_PER_TILE // CHUNK
NPAIR = NCHUNK // 2
CVEC = C // L


def _sc_body(xflat, ia, ib, ic, idd, wa, wb, wc, wd, out3,
             iav, ibv, icv, idv,
             ra0, rb0, rc0, rd0, wv0, ra1, rb1, rc1, rd1, wv1, ov0, ov1,
             sg0, sg1, so0, so1):
    wid = lax.axis_index("s") * NC + lax.axis_index("c")
    base0 = wid * PIX_PER_TILE

    tile = pl.ds(base0, PIX_PER_TILE)
    pltpu.sync_copy(ia.at[tile], iav)
    pltpu.sync_copy(ib.at[tile], ibv)
    pltpu.sync_copy(ic.at[tile], icv)
    pltpu.sync_copy(idd.at[tile], idv)

    slots = ((ra0, rb0, rc0, rd0, wv0, ov0, sg0, so0),
             (ra1, rb1, rc1, rd1, wv1, ov1, sg1, so1))

    bimg = wid // (NW // B)
    ibase = base0 - bimg * (H * W)

    def fire(g, slot):
        ra, rb, rc, rd, wv, _, sg, _ = slots[slot]
        sl = pl.ds(g * CHUNK, CHUNK)
        gsl = pl.ds(base0 + g * CHUNK, CHUNK)
        pltpu.async_copy(xflat.at[iav.at[sl]], ra, sg)
        pltpu.async_copy(xflat.at[ibv.at[sl]], rb, sg)
        pltpu.async_copy(xflat.at[icv.at[sl]], rc, sg)
        pltpu.async_copy(xflat.at[idv.at[sl]], rd, sg)
        pltpu.async_copy(wa.at[gsl], wv.at[0], sg)
        pltpu.async_copy(wb.at[gsl], wv.at[1], sg)
        pltpu.async_copy(wc.at[gsl], wv.at[2], sg)
        pltpu.async_copy(wd.at[gsl], wv.at[3], sg)

    def wait_gathers(slot):
        ra, rb, rc, rd, wv, _, sg, _ = slots[slot]
        dummy = xflat.at[pl.ds(0, CHUNK)]
        for dst in (ra, rb, rc, rd):
            pltpu.make_async_copy(dummy, dst, sg).wait()
        wdummy = wa.at[pl.ds(0, CHUNK)]
        for k in range(4):
            pltpu.make_async_copy(wdummy, wv.at[k], sg).wait()

    def drain_out(slot):
        ov, so = slots[slot][5], slots[slot][7]
        pltpu.make_async_copy(ov, out3.at[0, pl.ds(0, CHUNK)], so).wait()

    def compute_and_store(g, slot):
        ra, rb, rc, rd, wv, ov, _, so = slots[slot]
        off = g * CHUNK

        def grp(q, c2):
            woff = q * L
            wa16 = wv[0, pl.ds(woff, L)]
            wb16 = wv[1, pl.ds(woff, L)]
            wc16 = wv[2, pl.ds(woff, L)]
            wd16 = wv[3, pl.ds(woff, L)]
            for k in range(L):
                p = q * L + k
                w0, w1, w2, w3 = wa16[k], wb16[k], wc16[k], wd16[k]
                for j in range(CVEC):
                    s = pl.ds(j * L, L)
                    ov[p, s] = (ra[p, s] * w0 + rb[p, s] * w1
                                + rc[p, s] * w2 + rd[p, s] * w3)
            return c2

        lax.fori_loop(0, CHUNK // L, grp, 0)
        pltpu.async_copy(ov, out3.at[bimg, pl.ds(ibase + off, CHUNK)], so)

    fire(0, 0)

    def pair(i, carry):
        g0 = i * 2
        fire(g0 + 1, 1)
        wait_gathers(0)

        @pl.when(i > 0)
        def _():
            drain_out(0)

        compute_and_store(g0, 0)

        @pl.when(i < NPAIR - 1)
        def _():
            fire(g0 + 2, 0)

        wait_gathers(1)

        @pl.when(i > 0)
        def _():
            drain_out(1)

        compute_and_store(g0 + 1, 1)
        return carry

    lax.fori_loop(0, NPAIR, pair, 0)
    drain_out(0)
    drain_out(1)


@functools.partial(jax.jit, static_argnums=())
def _sc_gather(xflat, ia, ib, ic, idd, wa, wb, wc, wd):
    mesh = plsc.VectorSubcoreMesh(core_axis_name="c", subcore_axis_name="s")
    kern = functools.partial(
        pl.kernel,
        mesh=mesh,
        compiler_params=pltpu.CompilerParams(use_tc_tiling_on_sc=True),
        out_type=jax.ShapeDtypeStruct((B, H * W, C), jnp.float32),
        scratch_types=[
            pltpu.VMEM((PIX_PER_TILE,), jnp.int32),
            pltpu.VMEM((PIX_PER_TILE,), jnp.int32),
            pltpu.VMEM((PIX_PER_TILE,), jnp.int32),
            pltpu.VMEM((PIX_PER_TILE,), jnp.int32),
            pltpu.VMEM((CHUNK, CP), jnp.float32),
            pltpu.VMEM((CHUNK, CP), jnp.float32),
            pltpu.VMEM((CHUNK, CP), jnp.float32),
            pltpu.VMEM((CHUNK, CP), jnp.float32),
            pltpu.VMEM((4, CHUNK), jnp.float32),
            pltpu.VMEM((CHUNK, CP), jnp.float32),
            pltpu.VMEM((CHUNK, CP), jnp.float32),
            pltpu.VMEM((CHUNK, CP), jnp.float32),
            pltpu.VMEM((CHUNK, CP), jnp.float32),
            pltpu.VMEM((4, CHUNK), jnp.float32),
            pltpu.VMEM((CHUNK, C), jnp.float32),
            pltpu.VMEM((CHUNK, C), jnp.float32),
            pltpu.SemaphoreType.DMA,
            pltpu.SemaphoreType.DMA,
            pltpu.SemaphoreType.DMA,
            pltpu.SemaphoreType.DMA,
        ],
    )(_sc_body)
    return kern(xflat, ia, ib, ic, idd, wa, wb, wc, wd)


def kernel(X, W_loc, b_loc):
    xp, xpad = _pool(X)
    xp2 = xp.reshape(B, HP, WP * C)
    eye = jnp.eye(WP, dtype=jnp.float32)
    wbx = jnp.kron(eye, W_loc[:, 0:1])
    wby = jnp.kron(eye, W_loc[:, 1:2])
    bl = b_loc.reshape(1, 2)
    ia, ib, ic, idd, wa, wb, wc, wd = _coords(xp2, wbx, wby, bl)
    out3 = _sc_gather(
        xpad,
        ia.reshape(-1), ib.reshape(-1), ic.reshape(-1), idd.reshape(-1),
        wa.reshape(-1), wb.reshape(-1), wc.reshape(-1), wd.reshape(-1),
    )
    return out3.reshape(B, H, W, C)
